# baseline (speedup 1.0000x reference)
import functools

import jax
import jax.numpy as jnp
from jax import lax
from jax.experimental import pallas as pl
from jax.experimental.pallas import tpu as pltpu
from jax.experimental.pallas import tpu_sc as plsc

BATCH = 16384
N_ACTIONS = 64
NW = 32
BPW = BATCH // NW
IDX_W = 128
CH = BPW // IDX_W
LANES = 16


def _run(s_hbm, a_hbm, t_hbm, out_hbm, s_v, a_v, idx_v, val_v, sem):
    wid = lax.axis_index("s") * 2 + lax.axis_index("c")
    base = wid * BPW

    pltpu.sync_copy(s_hbm.at[pl.ds(base, BPW)], s_v)
    pltpu.sync_copy(a_hbm.at[pl.ds(base, BPW)], a_v)

    for i in range(BPW // LANES):
        sv = s_v[pl.ds(i * LANES, LANES)]
        av = a_v[pl.ds(i * LANES, LANES)]
        r, off = divmod(i * LANES, IDX_W)
        idx_v[r, pl.ds(off, LANES)] = sv * N_ACTIONS + av

    copies = [
        pltpu.async_copy(t_hbm.at[idx_v.at[c]], val_v.at[c], sem)
        for c in range(CH)
    ]
    for cp in copies:
        cp.wait()

    for c in range(CH):
        pltpu.sync_copy(val_v.at[c], out_hbm.at[pl.ds(base + c * IDX_W, IDX_W)])


def kernel(state, action, table):
    s = state.astype(jnp.int32)
    a = action.astype(jnp.int32)
    t_flat = table.reshape(-1)

    mesh = plsc.VectorSubcoreMesh(core_axis_name="c", subcore_axis_name="s")
    run = functools.partial(
        pl.kernel,
        mesh=mesh,
        out_type=jax.ShapeDtypeStruct((BATCH,), jnp.float32),
        scratch_types=[
            pltpu.VMEM((BPW,), jnp.int32),
            pltpu.VMEM((BPW,), jnp.int32),
            pltpu.VMEM((CH, IDX_W), jnp.int32),
            pltpu.VMEM((CH, IDX_W), jnp.float32),
            pltpu.SemaphoreType.DMA,
        ],
    )(_run)
    return run(s, a, t_flat)


# speedup vs baseline: 1.6493x; 1.6493x over previous
import functools

import jax
import jax.numpy as jnp
from jax import lax
from jax.experimental import pallas as pl
from jax.experimental.pallas import tpu as pltpu
from jax.experimental.pallas import tpu_sc as plsc

BATCH = 16384
N_ACTIONS = 64
NW = 32
BPW = BATCH // NW
LANES = 16
CHUNKS = BPW // LANES


def _run(s_hbm, a_hbm, t_hbm, out_hbm, s_v, a_v, rows_v, out_v, sem):
    wid = lax.axis_index("s") * 2 + lax.axis_index("c")
    base = wid * BPW

    pltpu.sync_copy(s_hbm.at[pl.ds(base, BPW)], s_v)
    pltpu.sync_copy(a_hbm.at[pl.ds(base, BPW)], a_v)

    copies = []
    for c in range(CHUNKS):
        sv = s_v[pl.ds(c * LANES, LANES)]
        gv = a_v[pl.ds(c * LANES, LANES)] // LANES
        for k in range(LANES):
            j = c * LANES + k
            copies.append(
                pltpu.async_copy(
                    t_hbm.at[sv[k], pl.ds(gv[k] * LANES, LANES)],
                    rows_v.at[pl.ds(j * LANES, LANES)], sem)
            )
    for cp in copies:
        cp.wait()

    for c in range(CHUNKS):
        av = a_v[pl.ds(c * LANES, LANES)]
        flat = (lax.iota(jnp.int32, LANES) + c * LANES) * LANES + av % LANES
        out_v[pl.ds(c * LANES, LANES)] = plsc.load_gather(rows_v, [flat])

    pltpu.sync_copy(out_v, out_hbm.at[pl.ds(base, BPW)])


def kernel(state, action, table):
    s = state.astype(jnp.int32)
    a = action.astype(jnp.int32)

    mesh = plsc.VectorSubcoreMesh(core_axis_name="c", subcore_axis_name="s")
    run = functools.partial(
        pl.kernel,
        mesh=mesh,
        compiler_params=pltpu.CompilerParams(needs_layout_passes=False),
        out_type=jax.ShapeDtypeStruct((BATCH,), jnp.float32),
        scratch_types=[
            pltpu.VMEM((BPW,), jnp.int32),
            pltpu.VMEM((BPW,), jnp.int32),
            pltpu.VMEM((BPW * LANES,), jnp.float32),
            pltpu.VMEM((BPW,), jnp.float32),
            pltpu.SemaphoreType.DMA,
        ],
    )(_run)
    return run(s, a, table)


# speedup vs baseline: 14.6115x; 8.8592x over previous
import functools

import jax
import jax.numpy as jnp
from jax import lax
from jax.experimental import pallas as pl
from jax.experimental.pallas import tpu as pltpu
from jax.experimental.pallas import tpu_sc as plsc

BATCH = 16384
N_ACTIONS = 64
NW = 32
BPW = BATCH // NW
LANES = 16
CHUNKS = BPW // LANES


def _run(s_hbm, a_hbm, t_hbm, out_hbm, s_v, a_v, gr_v, out_v, sem):
    wid = lax.axis_index("s") * 2 + lax.axis_index("c")
    base = wid * BPW

    pltpu.sync_copy(s_hbm.at[pl.ds(base, BPW)], s_v)
    pltpu.sync_copy(a_hbm.at[pl.ds(base, BPW)], a_v)

    copies = []
    for c in range(CHUNKS):
        sv = s_v[pl.ds(c * LANES, LANES)]
        av = a_v[pl.ds(c * LANES, LANES)]
        gv = sv // LANES
        for k in range(LANES):
            j = c * LANES + k
            copies.append(
                pltpu.async_copy(
                    t_hbm.at[av[k], pl.ds(gv[k] * LANES, LANES)],
                    gr_v.at[pl.ds(j * LANES, LANES)], sem)
            )
    for cp in copies:
        cp.wait()

    for c in range(CHUNKS):
        sv = s_v[pl.ds(c * LANES, LANES)]
        flat = (lax.iota(jnp.int32, LANES) + c * LANES) * LANES + sv % LANES
        out_v[pl.ds(c * LANES, LANES)] = plsc.load_gather(gr_v, [flat])

    pltpu.sync_copy(out_v, out_hbm.at[pl.ds(base, BPW)])


def kernel(state, action, table):
    s = state.astype(jnp.int32)
    a = action.astype(jnp.int32)
    t = table.T

    mesh = plsc.VectorSubcoreMesh(core_axis_name="c", subcore_axis_name="s")
    run = functools.partial(
        pl.kernel,
        mesh=mesh,
        compiler_params=pltpu.CompilerParams(needs_layout_passes=False),
        out_type=jax.ShapeDtypeStruct((BATCH,), jnp.float32),
        scratch_types=[
            pltpu.VMEM((BPW,), jnp.int32),
            pltpu.VMEM((BPW,), jnp.int32),
            pltpu.VMEM((BPW * LANES,), jnp.float32),
            pltpu.VMEM((BPW,), jnp.float32),
            pltpu.SemaphoreType.DMA,
        ],
    )(_run)
    return run(s, a, t)


# speedup vs baseline: 25.5964x; 1.7518x over previous
import functools

import jax
import jax.numpy as jnp
from jax import lax
from jax.experimental import pallas as pl
from jax.experimental.pallas import tpu as pltpu
from jax.experimental.pallas import tpu_sc as plsc

BATCH = 16384
N_ACTIONS = 64
NW = 32
BPW = BATCH // NW
LANES = 16
CHUNKS = BPW // LANES


def _run(s_hbm, a_hbm, t_hbm, out_hbm, s_v, a_v, gr_v, out_v, sem):
    wid = lax.axis_index("s") * 2 + lax.axis_index("c")
    base = wid * BPW

    pltpu.sync_copy(s_hbm.at[pl.ds(base, BPW)], s_v)
    pltpu.sync_copy(a_hbm.at[pl.ds(base, BPW)], a_v)

    def fetch_chunk(c, carry):
        sv = s_v[pl.ds(c * LANES, LANES)]
        av = a_v[pl.ds(c * LANES, LANES)]
        gv = sv // LANES
        for k in range(LANES):
            pltpu.async_copy(
                t_hbm.at[av[k], pl.ds(gv[k] * LANES, LANES)],
                gr_v.at[pl.ds((c * LANES + k) * LANES, LANES)], sem)
        return carry

    lax.fori_loop(0, CHUNKS, fetch_chunk, 0)

    pltpu.make_async_copy(
        out_hbm.at[pl.ds(0, BPW * LANES)], gr_v, sem).wait()

    def pick_chunk(c, carry):
        sv = s_v[pl.ds(c * LANES, LANES)]
        flat = (lax.iota(jnp.int32, LANES) + c * LANES) * LANES + sv % LANES
        out_v[pl.ds(c * LANES, LANES)] = plsc.load_gather(gr_v, [flat])
        return carry

    lax.fori_loop(0, CHUNKS, pick_chunk, 0)

    pltpu.sync_copy(out_v, out_hbm.at[pl.ds(base, BPW)])


def kernel(state, action, table):
    s = state.astype(jnp.int32)
    a = action.astype(jnp.int32)
    t = table.T

    mesh = plsc.VectorSubcoreMesh(core_axis_name="c", subcore_axis_name="s")
    run = functools.partial(
        pl.kernel,
        mesh=mesh,
        compiler_params=pltpu.CompilerParams(needs_layout_passes=False),
        out_type=jax.ShapeDtypeStruct((BATCH,), jnp.float32),
        scratch_types=[
            pltpu.VMEM((BPW,), jnp.int32),
            pltpu.VMEM((BPW,), jnp.int32),
            pltpu.VMEM((BPW * LANES,), jnp.float32),
            pltpu.VMEM((BPW,), jnp.float32),
            pltpu.SemaphoreType.DMA,
        ],
    )(_run)
    return run(s, a, t)
